# baseline (speedup 1.0000x reference)
import functools

import jax
import jax.numpy as jnp
from jax import lax
from jax.experimental import pallas as pl
from jax.experimental.pallas import tpu as pltpu
from jax.experimental.pallas import tpu_sc as plsc

D = 768
H = 12
DH = D // H
E = 8
K = 2
F = 3072
SB = 512
BT = 128
FB = 512
NF = F // FB


def _ln_rows(x, g, b):
    m = jnp.mean(x, axis=1, keepdims=True)
    xc = x - m
    v = jnp.mean(xc * xc, axis=1, keepdims=True)
    return xc * lax.rsqrt(v + 1e-5) * g + b


def _qkv_body(x_ref, g_ref, b_ref, wq_ref, bq_ref, wk_ref, bk_ref,
              wv_ref, bv_ref, q_ref, k_ref, v_ref):
    h = _ln_rows(x_ref[...], g_ref[...], b_ref[...]).astype(jnp.bfloat16)
    for w_ref, bb_ref, o_ref in ((wq_ref, bq_ref, q_ref),
                                 (wk_ref, bk_ref, k_ref),
                                 (wv_ref, bv_ref, v_ref)):
        r = jnp.dot(h, w_ref[...].astype(jnp.bfloat16),
                    preferred_element_type=jnp.float32) + bb_ref[...]
        rb = r.astype(jnp.bfloat16)
        for hh in range(H):
            o_ref[hh, :, :] = rb[:, hh * DH:(hh + 1) * DH]


def _qkv_call(x, ln1_g, ln1_b, Wq, bq, Wk, bk, Wv, bv, interpret=False):
    S = x.shape[0]
    hd = jax.ShapeDtypeStruct((H, S, DH), jnp.bfloat16)
    full = lambda shp: pl.BlockSpec(shp, lambda s: (0,) * len(shp))
    return pl.pallas_call(
        _qkv_body,
        grid=(S // SB,),
        in_specs=[
            pl.BlockSpec((SB, D), lambda s: (s, 0)),
            full((1, D)), full((1, D)),
            full((D, D)), full((1, D)),
            full((D, D)), full((1, D)),
            full((D, D)), full((1, D)),
        ],
        out_specs=[pl.BlockSpec((H, SB, DH), lambda s: (0, s, 0))] * 3,
        out_shape=[hd, hd, hd],
        compiler_params=pltpu.CompilerParams(
            dimension_semantics=("parallel",)),
        interpret=interpret,
    )(x, ln1_g, ln1_b, Wq, bq, Wk, bk, Wv, bv)


def _attn_body(x_ref, q_ref, k_ref, v_ref, wo_ref, bo_ref, g2_ref, b2_ref,
               wrt_ref, x2_ref, tb_ref, lgt_ref):
    hh = pl.program_id(1)
    q = q_ref[0]
    k = k_ref[0]
    v = v_ref[0]
    s = lax.dot_general(q, k, (((1,), (1,)), ((), ())),
                        preferred_element_type=jnp.float32) * (DH ** -0.5)
    s = s - jnp.max(s, axis=1, keepdims=True)
    p = jnp.exp(s)
    p = p / jnp.sum(p, axis=1, keepdims=True)
    o = jnp.dot(p.astype(jnp.bfloat16), v, preferred_element_type=jnp.float32)
    op = jnp.dot(o.astype(jnp.bfloat16), wo_ref[...].astype(jnp.bfloat16),
                 preferred_element_type=jnp.float32)

    @pl.when(hh == 0)
    def _():
        x2_ref[...] = x_ref[...] + bo_ref[...] + op

    @pl.when(hh > 0)
    def _():
        x2_ref[...] += op

    @pl.when(hh == H - 1)
    def _():
        t = _ln_rows(x2_ref[...], g2_ref[...], b2_ref[...])
        tb_ref[...] = t.astype(jnp.bfloat16)
        lgt_ref[...] = lax.dot_general(wrt_ref[...], t, (((1,), (1,)), ((), ())),
                                       preferred_element_type=jnp.float32)


def _attn_call(x, q, k, v, Wo, bo, ln2_g, ln2_b, WrT, interpret=False):
    S = x.shape[0]
    full = lambda shp: pl.BlockSpec(shp, lambda s, h: (0,) * len(shp))
    return pl.pallas_call(
        _attn_body,
        grid=(S // SB, H),
        in_specs=[
            pl.BlockSpec((SB, D), lambda s, h: (s, 0)),
            pl.BlockSpec((1, SB, DH), lambda s, h: (h, s, 0)),
            pl.BlockSpec((1, S, DH), lambda s, h: (h, 0, 0)),
            pl.BlockSpec((1, S, DH), lambda s, h: (h, 0, 0)),
            pl.BlockSpec((DH, D), lambda s, h: (h, 0)),
            full((1, D)), full((1, D)), full((1, D)),
            full((E, D)),
        ],
        out_specs=[
            pl.BlockSpec((SB, D), lambda s, h: (s, 0)),
            pl.BlockSpec((SB, D), lambda s, h: (s, 0)),
            pl.BlockSpec((E, SB), lambda s, h: (0, s)),
        ],
        out_shape=[
            jax.ShapeDtypeStruct((S, D), jnp.float32),
            jax.ShapeDtypeStruct((S, D), jnp.bfloat16),
            jax.ShapeDtypeStruct((E, S), jnp.float32),
        ],
        compiler_params=pltpu.CompilerParams(
            dimension_semantics=("parallel", "arbitrary")),
        interpret=interpret,
    )(x, q, k, v, Wo, bo, ln2_g, ln2_b, WrT)


def _moe_body(be_ref, st_ref, sg_ref, t_ref, w1_ref, b1_ref, w2_ref, b2_ref,
              y_ref, x_scr, acc_scr):
    S = t_ref.shape[0]
    f = pl.program_id(1)

    @pl.when(f == 0)
    def _():
        st = st_ref[...].astype(jnp.float32)
        iot = lax.broadcasted_iota(jnp.int32, (BT, S), 1).astype(jnp.float32)
        oh = jnp.where(st == iot,
                       jnp.float32(1), jnp.float32(0)).astype(jnp.bfloat16)
        x_scr[...] = jnp.dot(oh, t_ref[...],
                             preferred_element_type=jnp.float32).astype(jnp.bfloat16)
        acc_scr[...] = jnp.broadcast_to(b2_ref[0], (BT, D))

    h1 = jnp.dot(x_scr[...], w1_ref[0].astype(jnp.bfloat16),
                 preferred_element_type=jnp.float32) + b1_ref[0]
    h1 = 0.5 * h1 * (1.0 + lax.erf(h1 * (2 ** -0.5)))
    acc_scr[...] += jnp.dot(h1.astype(jnp.bfloat16), w2_ref[0].astype(jnp.bfloat16),
                            preferred_element_type=jnp.float32)

    @pl.when(f == NF - 1)
    def _():
        y_ref[...] = (acc_scr[...] * sg_ref[...]).astype(jnp.bfloat16)


def _moe_call(blk_e, st, sg, tbf, W1, b1, W2, b2, P, interpret=False):
    S = tbf.shape[0]
    NB = P // BT
    grid_spec = pltpu.PrefetchScalarGridSpec(
        num_scalar_prefetch=1,
        grid=(NB, NF),
        in_specs=[
            pl.BlockSpec((BT, 1), lambda b, f, be: (b, 0)),
            pl.BlockSpec((BT, 1), lambda b, f, be: (b, 0)),
            pl.BlockSpec((S, D), lambda b, f, be: (0, 0)),
            pl.BlockSpec((1, D, FB), lambda b, f, be: (be[b], 0, f)),
            pl.BlockSpec((1, 1, FB), lambda b, f, be: (be[b], 0, f)),
            pl.BlockSpec((1, FB, D), lambda b, f, be: (be[b], f, 0)),
            pl.BlockSpec((1, 1, D), lambda b, f, be: (be[b], 0, 0)),
        ],
        out_specs=pl.BlockSpec((BT, D), lambda b, f, be: (b, 0)),
        scratch_shapes=[
            pltpu.VMEM((BT, D), jnp.bfloat16),
            pltpu.VMEM((BT, D), jnp.float32),
        ],
    )
    return pl.pallas_call(
        _moe_body,
        grid_spec=grid_spec,
        out_shape=jax.ShapeDtypeStruct((P, D), jnp.bfloat16),
        compiler_params=pltpu.CompilerParams(
            dimension_semantics=("arbitrary", "arbitrary")),
        interpret=interpret,
    )(blk_e, st, sg, tbf, W1, b1, W2, b2)


_STAGE = 99
NSC = 16
TW = 2048 // NSC
NG = TW // 16


def _iota16():
    return lax.broadcasted_iota(jnp.int32, (16,), 0)


def _lane(vec, e):
    return jnp.sum(jnp.where(_iota16() == e, vec, 0))


def _route_body(lg_ref, st_ref, sg_ref, be_ref, lb_ref, pos_ref,
                lg_v, idx_v, gat_v, cnt_stage_v, pm_stage_v, cnt_all_v,
                pm_all_v, pos_v0, pos_v1, st_v, sg_v, posall_v, gatall_v,
                blk_v, lb_v, sh_cnt, sh_pm, sh_pos, sh_gat):
    @pl.when(lax.axis_index("c") == 0)
    def _core0():
        _route_core0(lg_ref, st_ref, sg_ref, be_ref, lb_ref, pos_ref,
                     lg_v, idx_v, gat_v, cnt_stage_v, pm_stage_v, cnt_all_v,
                     pm_all_v, pos_v0, pos_v1, st_v, sg_v, posall_v, gatall_v,
                     blk_v, lb_v, sh_cnt, sh_pm, sh_pos, sh_gat)


def _route_core0(lg_ref, st_ref, sg_ref, be_ref, lb_ref, pos_ref,
                 lg_v, idx_v, gat_v, cnt_stage_v, pm_stage_v, cnt_all_v,
                 pm_all_v, pos_v0, pos_v1, st_v, sg_v, posall_v, gatall_v,
                 blk_v, lb_v, sh_cnt, sh_pm, sh_pos, sh_gat):
    P = st_ref.shape[0]
    w = lax.axis_index("s")
    iota = _iota16()

    pltpu.sync_copy(lg_ref.at[w], lg_v)
    base = w * TW
    pma = [jnp.zeros((16,), jnp.float32) for _ in range(E)]
    cnt0 = jnp.zeros((16,), jnp.int32)
    cnt1 = jnp.zeros((16,), jnp.int32)
    for g in range(NG):
        le = [lg_v[e, pl.ds(g * 16, 16)] for e in range(E)]
        m1 = le[0]
        i1 = jnp.zeros((16,), jnp.int32)
        for e in range(1, E):
            upd = le[e] > m1
            m1 = jnp.where(upd, le[e], m1)
            i1 = jnp.where(upd, e, i1)
        m2 = jnp.full((16,), -3.0e38, jnp.float32)
        i2 = jnp.zeros((16,), jnp.int32)
        for e in range(E):
            upd = (le[e] > m2) & (i1 != e)
            m2 = jnp.where(upd, le[e], m2)
            i2 = jnp.where(upd, e, i2)
        pe = [jnp.exp(le[e] - m1) for e in range(E)]
        ssum = pe[0]
        for e in range(1, E):
            ssum = ssum + pe[e]
        rinv = 1.0 / ssum
        for e in range(E):
            pma[e] = pma[e] + pe[e] * rinv
        g1 = 1.0 / (1.0 + jnp.exp(m2 - m1))
        idx_v[0, pl.ds(g * 16, 16)] = i1
        idx_v[1, pl.ds(g * 16, 16)] = i2
        gat_v[0, pl.ds(g * 16, 16)] = g1
        gat_v[1, pl.ds(g * 16, 16)] = 1.0 - g1
        for e in range(E):
            cnt0 = cnt0 + jnp.where(iota == e,
                                    jnp.sum((i1 == e).astype(jnp.int32)), 0)
            cnt1 = cnt1 + jnp.where(iota == e,
                                    jnp.sum((i2 == e).astype(jnp.int32)), 0)
    cnt_stage_v[0, :] = cnt0
    cnt_stage_v[1, :] = cnt1
    pltpu.sync_copy(cnt_stage_v, sh_cnt.at[w])
    for e in range(E):
        pm_stage_v[e, :] = pma[e]
    pltpu.sync_copy(pm_stage_v, sh_pm.at[w])

    plsc.subcore_barrier()

    if _STAGE == 0:
        @pl.when(w == 0)
        def _():
            for g in range(P // 16):
                st_v[pl.ds(g * 16, 16)] = jnp.zeros((16,), jnp.int32)
                sg_v[pl.ds(g * 16, 16)] = jnp.zeros((16,), jnp.float32)
            pltpu.sync_copy(st_v, st_ref)
            pltpu.sync_copy(sg_v, sg_ref)
            for i in range(3):
                blk_v[pl.ds(i * 16, 16)] = jnp.zeros((16,), jnp.int32)
            pltpu.sync_copy(blk_v, be_ref)
            lb_v[...] = jnp.zeros((16,), jnp.float32)
            pltpu.sync_copy(lb_v, lb_ref)
        for g in range(NG):
            pos_v0[pl.ds(g * 16, 16)] = jnp.zeros((16,), jnp.int32)
            pos_v1[pl.ds(g * 16, 16)] = jnp.zeros((16,), jnp.int32)
        pltpu.sync_copy(pos_v0, pos_ref.at[0, pl.ds(base, TW)])
        pltpu.sync_copy(pos_v1, pos_ref.at[1, pl.ds(base, TW)])
        return

    pltpu.sync_copy(sh_cnt, cnt_all_v)
    tot = jnp.zeros((16,), jnp.int32)
    pref0 = jnp.zeros((16,), jnp.int32)
    pref1 = jnp.zeros((16,), jnp.int32)
    for i in range(NSC):
        c0r = cnt_all_v[i, 0, :]
        c1r = cnt_all_v[i, 1, :]
        both = c0r + c1r
        tot = tot + both
        before = jnp.where(i < w, 1, 0)
        pref0 = pref0 + both * before
        pref1 = pref1 + both * before + c0r * jnp.where(i == w, 1, 0)
    pc = lax.shift_left(lax.shift_right_logical(tot + (BT - 1), 7), 7)
    ic = plsc.cumsum(pc)
    pad_off = ic - pc

    carry = [pad_off + pref0, pad_off + pref1]
    for g in range(NG):
        for j, pos_v in ((0, pos_v0), (1, pos_v1)):
            eid = idx_v[j, pl.ds(g * 16, 16)]
            pos = jnp.zeros((16,), jnp.int32)
            for e in range(E):
                m = eid == e
                mi = m.astype(jnp.int32)
                cs = plsc.cumsum(mi)
                ce = _lane(carry[j], e)
                pos = jnp.where(m, ce + cs - 1, pos)
                carry[j] = carry[j] + jnp.where(iota == e, jnp.sum(mi), 0)
            pos_v[pl.ds(g * 16, 16)] = pos
    pltpu.sync_copy(pos_v0, sh_pos.at[0, pl.ds(base, TW)])
    pltpu.sync_copy(pos_v1, sh_pos.at[1, pl.ds(base, TW)])
    pltpu.sync_copy(gat_v.at[0], sh_gat.at[0, pl.ds(base, TW)])
    pltpu.sync_copy(gat_v.at[1], sh_gat.at[1, pl.ds(base, TW)])
    pltpu.sync_copy(pos_v0, pos_ref.at[0, pl.ds(base, TW)])
    pltpu.sync_copy(pos_v1, pos_ref.at[1, pl.ds(base, TW)])

    plsc.subcore_barrier()

    if _STAGE == 1:
        @pl.when(w == 0)
        def _():
            for g in range(P // 16):
                st_v[pl.ds(g * 16, 16)] = jnp.zeros((16,), jnp.int32)
                sg_v[pl.ds(g * 16, 16)] = jnp.zeros((16,), jnp.float32)
            pltpu.sync_copy(st_v, st_ref)
            pltpu.sync_copy(sg_v, sg_ref)
            for i in range(3):
                blk_v[pl.ds(i * 16, 16)] = jnp.zeros((16,), jnp.int32)
            pltpu.sync_copy(blk_v, be_ref)
            lb_v[...] = (cnt_all_v[0, 0, :] + cnt_all_v[0, 1, :]).astype(jnp.float32)
            pltpu.sync_copy(lb_v, lb_ref)
        return

    @pl.when(w == 0)
    def _():
        for g in range(P // 16):
            st_v[pl.ds(g * 16, 16)] = jnp.zeros((16,), jnp.int32)
            sg_v[pl.ds(g * 16, 16)] = jnp.zeros((16,), jnp.float32)
        pltpu.sync_copy(sh_pos, posall_v)
        pltpu.sync_copy(sh_gat, gatall_v)
        for j in range(2):
            for g in range(2048 // 16):
                pidx = posall_v[j, pl.ds(g * 16, 16)]
                plsc.store_scatter(st_v, [pidx], iota + g * 16)
                plsc.store_scatter(sg_v, [pidx], gatall_v[j, pl.ds(g * 16, 16)])
        pltpu.sync_copy(st_v, st_ref)
        pltpu.sync_copy(sg_v, sg_ref)
        for i in range(3):
            bid = iota + i * 16
            acc = jnp.zeros((16,), jnp.int32)
            for e in range(E):
                ge = _lane(ic, e)
                acc = acc + (bid * BT >= ge).astype(jnp.int32)
            blk_v[pl.ds(i * 16, 16)] = jnp.minimum(acc, E - 1)
        pltpu.sync_copy(blk_v, be_ref)
        pltpu.sync_copy(sh_pm, pm_all_v)
        lbterm = jnp.zeros((16,), jnp.float32)
        totf = tot.astype(jnp.float32)
        for e in range(E):
            pmv = jnp.zeros((16,), jnp.float32)
            for wv in range(NSC):
                pmv = pmv + pm_all_v[wv, e, :]
            fe = jnp.sum(jnp.where(iota == e, totf, jnp.float32(0)))
            lbterm = lbterm + pmv * fe
        lb = jnp.sum(lbterm) * jnp.float32(0.01 * E / (4096.0 * 2048.0))
        lb_v[...] = jnp.where(iota == 0, lb, jnp.float32(0))
        pltpu.sync_copy(lb_v, lb_ref)


def _route_call(lgT3, P, interpret=False):
    mesh = plsc.VectorSubcoreMesh(core_axis_name="c", subcore_axis_name="s")
    return pl.kernel(
        _route_body,
        out_type=(
            jax.ShapeDtypeStruct((P,), jnp.int32),
            jax.ShapeDtypeStruct((P,), jnp.float32),
            jax.ShapeDtypeStruct((48,), jnp.int32),
            jax.ShapeDtypeStruct((16,), jnp.float32),
            jax.ShapeDtypeStruct((2, 2048), jnp.int32),
        ),
        mesh=mesh,
        scratch_types=(
            pltpu.VMEM((E, TW), jnp.float32),
            pltpu.VMEM((2, TW), jnp.int32),
            pltpu.VMEM((2, TW), jnp.float32),
            pltpu.VMEM((2, 16), jnp.int32),
            pltpu.VMEM((E, 16), jnp.float32),
            pltpu.VMEM((NSC, 2, 16), jnp.int32),
            pltpu.VMEM((NSC, E, 16), jnp.float32),
            pltpu.VMEM((TW,), jnp.int32),
            pltpu.VMEM((TW,), jnp.int32),
            pltpu.VMEM((P,), jnp.int32),
            pltpu.VMEM((P,), jnp.float32),
            pltpu.VMEM((2, 2048), jnp.int32),
            pltpu.VMEM((2, 2048), jnp.float32),
            pltpu.VMEM((48,), jnp.int32),
            pltpu.VMEM((16,), jnp.float32),
            pltpu.VMEM_SHARED((NSC, 2, 16), jnp.int32),
            pltpu.VMEM_SHARED((NSC, E, 16), jnp.float32),
            pltpu.VMEM_SHARED((2, 2048), jnp.int32),
            pltpu.VMEM_SHARED((2, 2048), jnp.float32),
        ),
        compiler_params=pltpu.CompilerParams(needs_layout_passes=False),
        interpret=interpret,
    )(lgT3)


def _combine_body(p0_ref, p1_ref, x2_ref, y_ref, out_ref):
    P = y_ref.shape[0]
    i0 = p0_ref[...].astype(jnp.float32)
    i1 = p1_ref[...].astype(jnp.float32)
    iot = lax.broadcasted_iota(jnp.int32, (SB, P), 1).astype(jnp.float32)
    sel = (jnp.where(i0 == iot, jnp.float32(1), jnp.float32(0))
           + jnp.where(i1 == iot, jnp.float32(1), jnp.float32(0)))
    out_ref[...] = x2_ref[...] + jnp.dot(sel.astype(jnp.bfloat16), y_ref[...],
                                         preferred_element_type=jnp.float32)


def _combine_call(p0, p1, x2, y, interpret=False):
    S = x2.shape[0]
    P = y.shape[0]
    return pl.pallas_call(
        _combine_body,
        grid=(S // SB,),
        in_specs=[
            pl.BlockSpec((SB, 1), lambda s: (s, 0)),
            pl.BlockSpec((SB, 1), lambda s: (s, 0)),
            pl.BlockSpec((SB, D), lambda s: (s, 0)),
            pl.BlockSpec((P, D), lambda s: (0, 0)),
        ],
        out_specs=pl.BlockSpec((SB, D), lambda s: (s, 0)),
        out_shape=jax.ShapeDtypeStruct((S, D), jnp.float32),
        compiler_params=pltpu.CompilerParams(
            dimension_semantics=("parallel",)),
        interpret=interpret,
    )(p0, p1, x2, y)


def _route_jnp(lgT, N, P):
    logits = lgT.T
    probs = jax.nn.softmax(logits, axis=-1)
    i1 = jnp.argmax(probs, axis=-1)
    p1 = jnp.max(probs, axis=-1)
    masked = jnp.where(jax.nn.one_hot(i1, E, dtype=bool), -jnp.inf, probs)
    i2 = jnp.argmax(masked, axis=-1)
    p2 = jnp.max(masked, axis=-1)
    g1 = p1 / (p1 + p2)
    g2 = p2 / (p1 + p2)
    tw = N // NSC
    t_idx = jnp.arange(N).reshape(NSC, tw)
    eall = jnp.stack([i1.reshape(NSC, tw), i2.reshape(NSC, tw)], axis=1).reshape(-1)
    gall = jnp.stack([g1.reshape(NSC, tw), g2.reshape(NSC, tw)], axis=1).reshape(-1)
    tall = jnp.stack([t_idx, t_idx], axis=1).reshape(-1)
    oh = jax.nn.one_hot(eall, E, dtype=jnp.int32)
    counts = jnp.sum(oh, axis=0)
    pc = ((counts + BT - 1) // BT) * BT
    pad_end = jnp.cumsum(pc)
    pad_off = pad_end - pc
    rank = jnp.cumsum(oh, axis=0) - oh
    rank = jnp.take_along_axis(rank, eall[:, None], axis=1)[:, 0]
    pos = pad_off[eall] + rank
    slot_token = jnp.zeros((P,), jnp.int32).at[pos].set(tall)
    slot_gate = jnp.zeros((P,), jnp.float32).at[pos].set(gall)
    posr = pos.reshape(NSC, 2, tw)
    pos0 = posr[:, 0, :].reshape(N)
    pos1 = posr[:, 1, :].reshape(N)
    NBb = P // BT
    bstart = jnp.arange(NBb) * BT
    blk_e = jnp.sum((bstart[:, None] >= pad_end[None, :]).astype(jnp.int32), axis=1)
    blk_e = jnp.minimum(blk_e, E - 1)
    frac = counts.astype(jnp.float32) / (N * K)
    pmean = jnp.mean(probs, axis=0)
    lb = jnp.float32(0.01) * E * jnp.sum(frac * pmean)
    return slot_token, slot_gate, blk_e, pos0, pos1, lb


def kernel(x, ln1_g, ln1_b, ln2_g, ln2_b, Wq, bq, Wk, bk, Wv, bv, Wo, bo,
           Wr, W1, b1, W2, b2, interpret=False):
    B, S, _ = x.shape
    N = B * S
    P = K * N + E * BT
    x2d = x.reshape(N, D)
    r1 = lambda a: a.reshape(1, D)
    q, k, v = _qkv_call(x2d, r1(ln1_g), r1(ln1_b), Wq, r1(bq), Wk, r1(bk),
                        Wv, r1(bv), interpret=interpret)
    x2, tbf, lgT = _attn_call(x2d, q, k, v, Wo, r1(bo), r1(ln2_g), r1(ln2_b),
                              Wr.T, interpret=interpret)
    st, sg, be, pos0, pos1, lb = _route_jnp(lgT, N, P)
    y = _moe_call(be, st.reshape(P, 1), sg.reshape(P, 1), tbf,
                  W1, b1.reshape(E, 1, F), W2, b2.reshape(E, 1, D), P,
                  interpret=interpret)
    out = _combine_call(pos0.reshape(N, 1), pos1.reshape(N, 1), x2, y,
                        interpret=interpret).reshape(B, S, D)
    return out, lb


# speedup vs baseline: 1.0281x; 1.0281x over previous
import functools

import jax
import jax.numpy as jnp
from jax import lax
from jax.experimental import pallas as pl
from jax.experimental.pallas import tpu as pltpu
from jax.experimental.pallas import tpu_sc as plsc

D = 768
H = 12
DH = D // H
E = 8
K = 2
F = 3072
SB = 512
BT = 128
FB = 512
NF = F // FB


def _ln_rows(x, g, b):
    m = jnp.mean(x, axis=1, keepdims=True)
    xc = x - m
    v = jnp.mean(xc * xc, axis=1, keepdims=True)
    return xc * lax.rsqrt(v + 1e-5) * g + b


def _qkv_body(x_ref, g_ref, b_ref, wq_ref, bq_ref, wk_ref, bk_ref,
              wv_ref, bv_ref, q_ref, k_ref, v_ref):
    h = _ln_rows(x_ref[...], g_ref[...], b_ref[...]).astype(jnp.bfloat16)
    for w_ref, bb_ref, o_ref in ((wq_ref, bq_ref, q_ref),
                                 (wk_ref, bk_ref, k_ref),
                                 (wv_ref, bv_ref, v_ref)):
        r = jnp.dot(h, w_ref[...].astype(jnp.bfloat16),
                    preferred_element_type=jnp.float32) + bb_ref[...]
        rb = r.astype(jnp.bfloat16)
        for hh in range(H):
            o_ref[hh, :, :] = rb[:, hh * DH:(hh + 1) * DH]


def _qkv_call(x, ln1_g, ln1_b, Wq, bq, Wk, bk, Wv, bv, interpret=False):
    S = x.shape[0]
    hd = jax.ShapeDtypeStruct((H, S, DH), jnp.bfloat16)
    full = lambda shp: pl.BlockSpec(shp, lambda s: (0,) * len(shp))
    return pl.pallas_call(
        _qkv_body,
        grid=(S // SB,),
        in_specs=[
            pl.BlockSpec((SB, D), lambda s: (s, 0)),
            full((1, D)), full((1, D)),
            full((D, D)), full((1, D)),
            full((D, D)), full((1, D)),
            full((D, D)), full((1, D)),
        ],
        out_specs=[pl.BlockSpec((H, SB, DH), lambda s: (0, s, 0))] * 3,
        out_shape=[hd, hd, hd],
        compiler_params=pltpu.CompilerParams(
            dimension_semantics=("parallel",)),
        interpret=interpret,
    )(x, ln1_g, ln1_b, Wq, bq, Wk, bk, Wv, bv)


def _attn_body(x_ref, q_ref, k_ref, v_ref, wo_ref, bo_ref, g2_ref, b2_ref,
               wrt_ref, x2_ref, tb_ref, lgt_ref):
    hh = pl.program_id(1)
    q = q_ref[0]
    k = k_ref[0]
    v = v_ref[0]
    s = lax.dot_general(q, k, (((1,), (1,)), ((), ())),
                        preferred_element_type=jnp.float32) * (DH ** -0.5)
    s = s - jnp.max(s, axis=1, keepdims=True)
    p = jnp.exp(s)
    p = p / jnp.sum(p, axis=1, keepdims=True)
    o = jnp.dot(p.astype(jnp.bfloat16), v, preferred_element_type=jnp.float32)
    op = jnp.dot(o.astype(jnp.bfloat16), wo_ref[...].astype(jnp.bfloat16),
                 preferred_element_type=jnp.float32)

    @pl.when(hh == 0)
    def _():
        x2_ref[...] = x_ref[...] + bo_ref[...] + op

    @pl.when(hh > 0)
    def _():
        x2_ref[...] += op

    @pl.when(hh == H - 1)
    def _():
        t = _ln_rows(x2_ref[...], g2_ref[...], b2_ref[...])
        tb_ref[...] = t.astype(jnp.bfloat16)
        lgt_ref[...] = lax.dot_general(wrt_ref[...], t, (((1,), (1,)), ((), ())),
                                       preferred_element_type=jnp.float32)


def _attn_call(x, q, k, v, Wo, bo, ln2_g, ln2_b, WrT, interpret=False):
    S = x.shape[0]
    full = lambda shp: pl.BlockSpec(shp, lambda s, h: (0,) * len(shp))
    return pl.pallas_call(
        _attn_body,
        grid=(S // SB, H),
        in_specs=[
            pl.BlockSpec((SB, D), lambda s, h: (s, 0)),
            pl.BlockSpec((1, SB, DH), lambda s, h: (h, s, 0)),
            pl.BlockSpec((1, S, DH), lambda s, h: (h, 0, 0)),
            pl.BlockSpec((1, S, DH), lambda s, h: (h, 0, 0)),
            pl.BlockSpec((DH, D), lambda s, h: (h, 0)),
            full((1, D)), full((1, D)), full((1, D)),
            full((E, D)),
        ],
        out_specs=[
            pl.BlockSpec((SB, D), lambda s, h: (s, 0)),
            pl.BlockSpec((SB, D), lambda s, h: (s, 0)),
            pl.BlockSpec((E, SB), lambda s, h: (0, s)),
        ],
        out_shape=[
            jax.ShapeDtypeStruct((S, D), jnp.float32),
            jax.ShapeDtypeStruct((S, D), jnp.bfloat16),
            jax.ShapeDtypeStruct((E, S), jnp.float32),
        ],
        compiler_params=pltpu.CompilerParams(
            dimension_semantics=("parallel", "arbitrary")),
        interpret=interpret,
    )(x, q, k, v, Wo, bo, ln2_g, ln2_b, WrT)


def _moe_body(be_ref, st_ref, sg_ref, t_ref, w1_ref, b1_ref, w2_ref, b2_ref,
              y_ref, x_scr, acc_scr):
    S = t_ref.shape[0]
    f = pl.program_id(1)

    @pl.when(f == 0)
    def _():
        st = st_ref[...].astype(jnp.float32)
        iot = lax.broadcasted_iota(jnp.int32, (BT, S), 1).astype(jnp.float32)
        oh = jnp.where(st == iot,
                       jnp.float32(1), jnp.float32(0)).astype(jnp.bfloat16)
        x_scr[...] = jnp.dot(oh, t_ref[...],
                             preferred_element_type=jnp.float32).astype(jnp.bfloat16)
        acc_scr[...] = jnp.broadcast_to(b2_ref[0], (BT, D))

    h1 = jnp.dot(x_scr[...], w1_ref[0].astype(jnp.bfloat16),
                 preferred_element_type=jnp.float32) + b1_ref[0]
    h1 = 0.5 * h1 * (1.0 + lax.erf(h1 * (2 ** -0.5)))
    acc_scr[...] += jnp.dot(h1.astype(jnp.bfloat16), w2_ref[0].astype(jnp.bfloat16),
                            preferred_element_type=jnp.float32)

    @pl.when(f == NF - 1)
    def _():
        y_ref[...] = (acc_scr[...] * sg_ref[...]).astype(jnp.bfloat16)


def _moe_call(blk_e, st, sg, tbf, W1, b1, W2, b2, P, interpret=False):
    S = tbf.shape[0]
    NB = P // BT
    grid_spec = pltpu.PrefetchScalarGridSpec(
        num_scalar_prefetch=1,
        grid=(NB, NF),
        in_specs=[
            pl.BlockSpec((BT, 1), lambda b, f, be: (b, 0)),
            pl.BlockSpec((BT, 1), lambda b, f, be: (b, 0)),
            pl.BlockSpec((S, D), lambda b, f, be: (0, 0)),
            pl.BlockSpec((1, D, FB), lambda b, f, be: (be[b], 0, f)),
            pl.BlockSpec((1, 1, FB), lambda b, f, be: (be[b], 0, f)),
            pl.BlockSpec((1, FB, D), lambda b, f, be: (be[b], f, 0)),
            pl.BlockSpec((1, 1, D), lambda b, f, be: (be[b], 0, 0)),
        ],
        out_specs=pl.BlockSpec((BT, D), lambda b, f, be: (b, 0)),
        scratch_shapes=[
            pltpu.VMEM((BT, D), jnp.bfloat16),
            pltpu.VMEM((BT, D), jnp.float32),
        ],
    )
    return pl.pallas_call(
        _moe_body,
        grid_spec=grid_spec,
        out_shape=jax.ShapeDtypeStruct((P, D), jnp.bfloat16),
        compiler_params=pltpu.CompilerParams(
            dimension_semantics=("arbitrary", "arbitrary")),
        interpret=interpret,
    )(blk_e, st, sg, tbf, W1, b1, W2, b2)


_STAGE = 99
NSC = 16
TW = 2048 // NSC
NG = TW // 16


def _iota16():
    return lax.broadcasted_iota(jnp.int32, (16,), 0)


def _lane(vec, e):
    return jnp.sum(jnp.where(_iota16() == e, vec, 0))


def _route_body(lg_ref, st_ref, sg_ref, be_ref, lb_ref, pos_ref,
                lg_v, idx_v, gat_v, cnt_stage_v, pm_stage_v, cnt_all_v,
                pm_all_v, pos_v0, pos_v1, st_v, sg_v, posall_v, gatall_v,
                blk_v, lb_v, sh_cnt, sh_pm, sh_pos, sh_gat):
    @pl.when(lax.axis_index("c") == 0)
    def _core0():
        _route_core0(lg_ref, st_ref, sg_ref, be_ref, lb_ref, pos_ref,
                     lg_v, idx_v, gat_v, cnt_stage_v, pm_stage_v, cnt_all_v,
                     pm_all_v, pos_v0, pos_v1, st_v, sg_v, posall_v, gatall_v,
                     blk_v, lb_v, sh_cnt, sh_pm, sh_pos, sh_gat)


def _route_core0(lg_ref, st_ref, sg_ref, be_ref, lb_ref, pos_ref,
                 lg_v, idx_v, gat_v, cnt_stage_v, pm_stage_v, cnt_all_v,
                 pm_all_v, pos_v0, pos_v1, st_v, sg_v, posall_v, gatall_v,
                 blk_v, lb_v, sh_cnt, sh_pm, sh_pos, sh_gat):
    P = st_ref.shape[0]
    w = lax.axis_index("s")
    iota = _iota16()

    pltpu.sync_copy(lg_ref.at[w], lg_v)
    base = w * TW
    pma = [jnp.zeros((16,), jnp.float32) for _ in range(E)]
    cnt0 = jnp.zeros((16,), jnp.int32)
    cnt1 = jnp.zeros((16,), jnp.int32)
    for g in range(NG):
        le = [lg_v[e, pl.ds(g * 16, 16)] for e in range(E)]
        m1 = le[0]
        i1 = jnp.zeros((16,), jnp.int32)
        for e in range(1, E):
            upd = le[e] > m1
            m1 = jnp.where(upd, le[e], m1)
            i1 = jnp.where(upd, e, i1)
        m2 = jnp.full((16,), -3.0e38, jnp.float32)
        i2 = jnp.zeros((16,), jnp.int32)
        for e in range(E):
            upd = (le[e] > m2) & (i1 != e)
            m2 = jnp.where(upd, le[e], m2)
            i2 = jnp.where(upd, e, i2)
        pe = [jnp.exp(le[e] - m1) for e in range(E)]
        ssum = pe[0]
        for e in range(1, E):
            ssum = ssum + pe[e]
        rinv = 1.0 / ssum
        for e in range(E):
            pma[e] = pma[e] + pe[e] * rinv
        g1 = 1.0 / (1.0 + jnp.exp(m2 - m1))
        idx_v[0, pl.ds(g * 16, 16)] = i1
        idx_v[1, pl.ds(g * 16, 16)] = i2
        gat_v[0, pl.ds(g * 16, 16)] = g1
        gat_v[1, pl.ds(g * 16, 16)] = 1.0 - g1
        for e in range(E):
            cnt0 = cnt0 + jnp.where(iota == e,
                                    jnp.sum((i1 == e).astype(jnp.int32)), 0)
            cnt1 = cnt1 + jnp.where(iota == e,
                                    jnp.sum((i2 == e).astype(jnp.int32)), 0)
    cnt_stage_v[0, :] = cnt0
    cnt_stage_v[1, :] = cnt1
    pltpu.sync_copy(cnt_stage_v, sh_cnt.at[w])
    for e in range(E):
        pm_stage_v[e, :] = pma[e]
    pltpu.sync_copy(pm_stage_v, sh_pm.at[w])

    plsc.subcore_barrier()

    if _STAGE == 0:
        @pl.when(w == 0)
        def _():
            for g in range(P // 16):
                st_v[pl.ds(g * 16, 16)] = jnp.zeros((16,), jnp.int32)
                sg_v[pl.ds(g * 16, 16)] = jnp.zeros((16,), jnp.float32)
            pltpu.sync_copy(st_v, st_ref)
            pltpu.sync_copy(sg_v, sg_ref)
            for i in range(3):
                blk_v[pl.ds(i * 16, 16)] = jnp.zeros((16,), jnp.int32)
            pltpu.sync_copy(blk_v, be_ref)
            lb_v[...] = jnp.zeros((16,), jnp.float32)
            pltpu.sync_copy(lb_v, lb_ref)
        for g in range(NG):
            pos_v0[pl.ds(g * 16, 16)] = jnp.zeros((16,), jnp.int32)
            pos_v1[pl.ds(g * 16, 16)] = jnp.zeros((16,), jnp.int32)
        pltpu.sync_copy(pos_v0, pos_ref.at[0, pl.ds(base, TW)])
        pltpu.sync_copy(pos_v1, pos_ref.at[1, pl.ds(base, TW)])
        return

    pltpu.sync_copy(sh_cnt, cnt_all_v)
    tot = jnp.zeros((16,), jnp.int32)
    pref0 = jnp.zeros((16,), jnp.int32)
    pref1 = jnp.zeros((16,), jnp.int32)
    for i in range(NSC):
        c0r = cnt_all_v[i, 0, :]
        c1r = cnt_all_v[i, 1, :]
        both = c0r + c1r
        tot = tot + both
        before = jnp.where(i < w, 1, 0)
        pref0 = pref0 + both * before
        pref1 = pref1 + both * before + c0r * jnp.where(i == w, 1, 0)
    pc = lax.shift_left(lax.shift_right_logical(tot + (BT - 1), 7), 7)
    ic = plsc.cumsum(pc)
    pad_off = ic - pc

    carry = [pad_off + pref0, pad_off + pref1]
    for g in range(NG):
        for j, pos_v in ((0, pos_v0), (1, pos_v1)):
            eid = idx_v[j, pl.ds(g * 16, 16)]
            pos = jnp.zeros((16,), jnp.int32)
            for e in range(E):
                m = eid == e
                mi = m.astype(jnp.int32)
                cs = plsc.cumsum(mi)
                ce = _lane(carry[j], e)
                pos = jnp.where(m, ce + cs - 1, pos)
                carry[j] = carry[j] + jnp.where(iota == e, jnp.sum(mi), 0)
            pos_v[pl.ds(g * 16, 16)] = pos
    pltpu.sync_copy(pos_v0, sh_pos.at[0, pl.ds(base, TW)])
    pltpu.sync_copy(pos_v1, sh_pos.at[1, pl.ds(base, TW)])
    pltpu.sync_copy(gat_v.at[0], sh_gat.at[0, pl.ds(base, TW)])
    pltpu.sync_copy(gat_v.at[1], sh_gat.at[1, pl.ds(base, TW)])
    pltpu.sync_copy(pos_v0, pos_ref.at[0, pl.ds(base, TW)])
    pltpu.sync_copy(pos_v1, pos_ref.at[1, pl.ds(base, TW)])

    plsc.subcore_barrier()

    if _STAGE == 1:
        @pl.when(w == 0)
        def _():
            for g in range(P // 16):
                st_v[pl.ds(g * 16, 16)] = jnp.zeros((16,), jnp.int32)
                sg_v[pl.ds(g * 16, 16)] = jnp.zeros((16,), jnp.float32)
            pltpu.sync_copy(st_v, st_ref)
            pltpu.sync_copy(sg_v, sg_ref)
            for i in range(3):
                blk_v[pl.ds(i * 16, 16)] = jnp.zeros((16,), jnp.int32)
            pltpu.sync_copy(blk_v, be_ref)
            lb_v[...] = (cnt_all_v[0, 0, :] + cnt_all_v[0, 1, :]).astype(jnp.float32)
            pltpu.sync_copy(lb_v, lb_ref)
        return

    @pl.when(w == 0)
    def _():
        for g in range(P // 16):
            st_v[pl.ds(g * 16, 16)] = jnp.zeros((16,), jnp.int32)
            sg_v[pl.ds(g * 16, 16)] = jnp.zeros((16,), jnp.float32)
        pltpu.sync_copy(sh_pos, posall_v)
        pltpu.sync_copy(sh_gat, gatall_v)
        for j in range(2):
            for g in range(2048 // 16):
                pidx = posall_v[j, pl.ds(g * 16, 16)]
                plsc.store_scatter(st_v, [pidx], iota + g * 16)
                plsc.store_scatter(sg_v, [pidx], gatall_v[j, pl.ds(g * 16, 16)])
        pltpu.sync_copy(st_v, st_ref)
        pltpu.sync_copy(sg_v, sg_ref)
        for i in range(3):
            bid = iota + i * 16
            acc = jnp.zeros((16,), jnp.int32)
            for e in range(E):
                ge = _lane(ic, e)
                acc = acc + (bid * BT >= ge).astype(jnp.int32)
            blk_v[pl.ds(i * 16, 16)] = jnp.minimum(acc, E - 1)
        pltpu.sync_copy(blk_v, be_ref)
        pltpu.sync_copy(sh_pm, pm_all_v)
        lbterm = jnp.zeros((16,), jnp.float32)
        totf = tot.astype(jnp.float32)
        for e in range(E):
            pmv = jnp.zeros((16,), jnp.float32)
            for wv in range(NSC):
                pmv = pmv + pm_all_v[wv, e, :]
            fe = jnp.sum(jnp.where(iota == e, totf, jnp.float32(0)))
            lbterm = lbterm + pmv * fe
        lb = jnp.sum(lbterm) * jnp.float32(0.01 * E / (4096.0 * 2048.0))
        lb_v[...] = jnp.where(iota == 0, lb, jnp.float32(0))
        pltpu.sync_copy(lb_v, lb_ref)


def _route_call(lgT3, P, interpret=False):
    mesh = plsc.VectorSubcoreMesh(core_axis_name="c", subcore_axis_name="s")
    return pl.kernel(
        _route_body,
        out_type=(
            jax.ShapeDtypeStruct((P,), jnp.int32),
            jax.ShapeDtypeStruct((P,), jnp.float32),
            jax.ShapeDtypeStruct((48,), jnp.int32),
            jax.ShapeDtypeStruct((16,), jnp.float32),
            jax.ShapeDtypeStruct((2, 2048), jnp.int32),
        ),
        mesh=mesh,
        scratch_types=(
            pltpu.VMEM((E, TW), jnp.float32),
            pltpu.VMEM((2, TW), jnp.int32),
            pltpu.VMEM((2, TW), jnp.float32),
            pltpu.VMEM((2, 16), jnp.int32),
            pltpu.VMEM((E, 16), jnp.float32),
            pltpu.VMEM((NSC, 2, 16), jnp.int32),
            pltpu.VMEM((NSC, E, 16), jnp.float32),
            pltpu.VMEM((TW,), jnp.int32),
            pltpu.VMEM((TW,), jnp.int32),
            pltpu.VMEM((P,), jnp.int32),
            pltpu.VMEM((P,), jnp.float32),
            pltpu.VMEM((2, 2048), jnp.int32),
            pltpu.VMEM((2, 2048), jnp.float32),
            pltpu.VMEM((48,), jnp.int32),
            pltpu.VMEM((16,), jnp.float32),
            pltpu.VMEM_SHARED((NSC, 2, 16), jnp.int32),
            pltpu.VMEM_SHARED((NSC, E, 16), jnp.float32),
            pltpu.VMEM_SHARED((2, 2048), jnp.int32),
            pltpu.VMEM_SHARED((2, 2048), jnp.float32),
        ),
        compiler_params=pltpu.CompilerParams(needs_layout_passes=False),
        interpret=interpret,
    )(lgT3)


def _slot_body(pa_ref, val_ref, st_ref, sg_ref):
    A = pa_ref.shape[1]
    b = pl.program_id(0)
    iot = lax.broadcasted_iota(jnp.int32, (SB, A), 0) + b * SB
    oh = jnp.where(pa_ref[...] == iot.astype(jnp.float32),
                   jnp.float32(1), jnp.float32(0))
    r = jnp.dot(oh, val_ref[...], preferred_element_type=jnp.float32)
    st_ref[...] = r[:, 0:1].astype(jnp.int32)
    sg_ref[...] = r[:, 1:2]


def _slot_call(pos_all, vals, P, interpret=False):
    A = pos_all.shape[1]
    return pl.pallas_call(
        _slot_body,
        grid=(P // SB,),
        in_specs=[
            pl.BlockSpec((1, A), lambda b: (0, 0)),
            pl.BlockSpec((A, 2), lambda b: (0, 0)),
        ],
        out_specs=[
            pl.BlockSpec((SB, 1), lambda b: (b, 0)),
            pl.BlockSpec((SB, 1), lambda b: (b, 0)),
        ],
        out_shape=[
            jax.ShapeDtypeStruct((P, 1), jnp.int32),
            jax.ShapeDtypeStruct((P, 1), jnp.float32),
        ],
        compiler_params=pltpu.CompilerParams(
            dimension_semantics=("parallel",)),
        interpret=interpret,
    )(pos_all, vals)


def _combine_body(p0_ref, p1_ref, x2_ref, y_ref, out_ref):
    P = y_ref.shape[0]
    i0 = p0_ref[...].astype(jnp.float32)
    i1 = p1_ref[...].astype(jnp.float32)
    iot = lax.broadcasted_iota(jnp.int32, (SB, P), 1).astype(jnp.float32)
    sel = (jnp.where(i0 == iot, jnp.float32(1), jnp.float32(0))
           + jnp.where(i1 == iot, jnp.float32(1), jnp.float32(0)))
    out_ref[...] = x2_ref[...] + jnp.dot(sel.astype(jnp.bfloat16), y_ref[...],
                                         preferred_element_type=jnp.float32)


def _combine_call(p0, p1, x2, y, interpret=False):
    S = x2.shape[0]
    P = y.shape[0]
    return pl.pallas_call(
        _combine_body,
        grid=(S // SB,),
        in_specs=[
            pl.BlockSpec((SB, 1), lambda s: (s, 0)),
            pl.BlockSpec((SB, 1), lambda s: (s, 0)),
            pl.BlockSpec((SB, D), lambda s: (s, 0)),
            pl.BlockSpec((P, D), lambda s: (0, 0)),
        ],
        out_specs=pl.BlockSpec((SB, D), lambda s: (s, 0)),
        out_shape=jax.ShapeDtypeStruct((S, D), jnp.float32),
        compiler_params=pltpu.CompilerParams(
            dimension_semantics=("parallel",)),
        interpret=interpret,
    )(p0, p1, x2, y)


def _route_jnp(lgT, N, P):
    logits = lgT.T
    probs = jax.nn.softmax(logits, axis=-1)
    i1 = jnp.argmax(probs, axis=-1)
    p1 = jnp.max(probs, axis=-1)
    masked = jnp.where(jax.nn.one_hot(i1, E, dtype=bool), -jnp.inf, probs)
    i2 = jnp.argmax(masked, axis=-1)
    p2 = jnp.max(masked, axis=-1)
    g1 = p1 / (p1 + p2)
    g2 = p2 / (p1 + p2)
    tw = N // NSC
    t_idx = jnp.arange(N).reshape(NSC, tw)
    eall = jnp.stack([i1.reshape(NSC, tw), i2.reshape(NSC, tw)], axis=1).reshape(-1)
    gall = jnp.stack([g1.reshape(NSC, tw), g2.reshape(NSC, tw)], axis=1).reshape(-1)
    tall = jnp.stack([t_idx, t_idx], axis=1).reshape(-1)
    oh = jax.nn.one_hot(eall, E, dtype=jnp.int32)
    counts = jnp.sum(oh, axis=0)
    pc = ((counts + BT - 1) // BT) * BT
    pad_end = jnp.cumsum(pc)
    pad_off = pad_end - pc
    rank = jnp.cumsum(oh, axis=0) - oh
    rank = jnp.take_along_axis(rank, eall[:, None], axis=1)[:, 0]
    pos = pad_off[eall] + rank
    vals = jnp.stack([tall.astype(jnp.float32), gall], axis=1)
    posr = pos.reshape(NSC, 2, tw)
    pos0 = posr[:, 0, :].reshape(N)
    pos1 = posr[:, 1, :].reshape(N)
    NBb = P // BT
    bstart = jnp.arange(NBb) * BT
    blk_e = jnp.sum((bstart[:, None] >= pad_end[None, :]).astype(jnp.int32), axis=1)
    blk_e = jnp.minimum(blk_e, E - 1)
    frac = counts.astype(jnp.float32) / (N * K)
    pmean = jnp.mean(probs, axis=0)
    lb = jnp.float32(0.01) * E * jnp.sum(frac * pmean)
    return pos.astype(jnp.float32).reshape(1, -1), vals, blk_e, pos0, pos1, lb


def kernel(x, ln1_g, ln1_b, ln2_g, ln2_b, Wq, bq, Wk, bk, Wv, bv, Wo, bo,
           Wr, W1, b1, W2, b2, interpret=False):
    B, S, _ = x.shape
    N = B * S
    P = K * N + E * BT
    x2d = x.reshape(N, D)
    r1 = lambda a: a.reshape(1, D)
    q, k, v = _qkv_call(x2d, r1(ln1_g), r1(ln1_b), Wq, r1(bq), Wk, r1(bk),
                        Wv, r1(bv), interpret=interpret)
    x2, tbf, lgT = _attn_call(x2d, q, k, v, Wo, r1(bo), r1(ln2_g), r1(ln2_b),
                              Wr.T, interpret=interpret)
    pos_all, vals, be, pos0, pos1, lb = _route_jnp(lgT, N, P)
    st, sg = _slot_call(pos_all, vals, P, interpret=interpret)
    y = _moe_call(be, st, sg, tbf,
                  W1, b1.reshape(E, 1, F), W2, b2.reshape(E, 1, D), P,
                  interpret=interpret)
    out = _combine_call(pos0.reshape(N, 1), pos1.reshape(N, 1), x2, y,
                        interpret=interpret).reshape(B, S, D)
    return out, lb
